# baseline (speedup 1.0000x reference)
import functools

import jax
import jax.numpy as jnp
from jax import lax
from jax.experimental import pallas as pl
from jax.experimental.pallas import tpu as pltpu
from jax.experimental.pallas import tpu_sc as plsc

N_NODES = 10000
D = 128
E = 320000

NC = 2
NS = 16
NW = NC * NS
CHUNK = 128
EPW = E // NW
NCH = 80
EPW_PAD = NCH * CHUNK
E_PAD = EPW_PAD * NW
NCHH = NCH // 2
GRP = 8
NGRP = NCHH // GRP
N_PAD = 10240
DST_PAD = 10008
RPT = N_PAD // NS

@functools.cache
def _sc_kernels():
    mesh = plsc.VectorSubcoreMesh(core_axis_name="c", subcore_axis_name="s",
                                  num_cores=NC, num_subcores=NS)

    @functools.partial(
        pl.kernel,
        out_type=jax.ShapeDtypeStruct((NC, N_PAD), jnp.float32),
        mesh=mesh,
        scratch_types=[
            pltpu.VMEM((NCH, CHUNK), jnp.int32),
            pltpu.VMEM((CHUNK,), jnp.float32),
            pltpu.VMEM_SHARED((N_PAD,), jnp.float32),
        ],
    )
    def sc_deg(dst_hbm, zeros1_hbm, out_hbm, dst_v, ones_v, deg_sh):
        c = lax.axis_index("c")
        s = lax.axis_index("s")
        wid = s * NC + c
        for i in range(CHUNK // 16):
            ones_v[pl.ds(i * 16, 16)] = jnp.ones((16,), jnp.float32)
        pltpu.sync_copy(zeros1_hbm.at[pl.ds(s * RPT, RPT)],
                        deg_sh.at[pl.ds(s * RPT, RPT)])
        plsc.subcore_barrier()
        pltpu.sync_copy(dst_hbm.at[wid], dst_v)

        def body(j, carry):
            pltpu.sync_copy(ones_v, deg_sh.at[dst_v.at[j]], add=True)
            return carry

        lax.fori_loop(0, NCH, body, 0)
        plsc.subcore_barrier()
        pltpu.sync_copy(deg_sh.at[pl.ds(s * RPT, RPT)],
                        out_hbm.at[c, pl.ds(s * RPT, RPT)])

    @functools.partial(
        pl.kernel,
        out_type=jax.ShapeDtypeStruct((NC, N_PAD, D), jnp.float32),
        mesh=mesh,
        scratch_types=[
            pltpu.VMEM((NCH, CHUNK), jnp.int32),
            pltpu.VMEM((NCH, CHUNK), jnp.int32),
            pltpu.VMEM((CHUNK, D), jnp.float32),
            pltpu.VMEM_SHARED((N_PAD, D), jnp.float32),
            pltpu.SemaphoreType.DMA,
        ],
    )
    def sc_msg(src_hbm, dst_hbm, y_hbm, zeros2_hbm, out_hbm,
               src_v, dst_v, rows_v, agg_sh, gsem):
        c = lax.axis_index("c")
        s = lax.axis_index("s")
        wid = s * NC + c
        pltpu.sync_copy(zeros2_hbm.at[pl.ds(s * RPT, RPT)],
                        agg_sh.at[pl.ds(s * RPT, RPT)])
        pltpu.sync_copy(src_hbm.at[wid], src_v)
        pltpu.sync_copy(dst_hbm.at[wid], dst_v)
        plsc.subcore_barrier()

        def body(j, carry):
            pltpu.async_copy(y_hbm.at[src_v.at[j]], rows_v, gsem).wait()
            pltpu.sync_copy(rows_v, agg_sh.at[dst_v.at[j]], add=True)
            return carry

        lax.fori_loop(0, NCH, body, 0)
        plsc.subcore_barrier()
        pltpu.sync_copy(agg_sh.at[pl.ds(s * RPT, RPT)],
                        out_hbm.at[c, pl.ds(s * RPT, RPT)])

    return sc_deg, sc_msg


_R = 2000
_GRID = N_NODES // _R


def _tc_in(x, W0, b0r):
    def body(x_ref, w_ref, b_ref, o_ref):
        h = jnp.dot(x_ref[...], w_ref[...], preferred_element_type=jnp.float32)
        o_ref[...] = jnp.maximum(h + b_ref[...], 0.0)

    return pl.pallas_call(
        body,
        grid=(_GRID,),
        in_specs=[
            pl.BlockSpec((_R, D), lambda i: (i, 0)),
            pl.BlockSpec((D, D), lambda i: (0, 0)),
            pl.BlockSpec((1, D), lambda i: (0, 0)),
        ],
        out_specs=pl.BlockSpec((_R, D), lambda i: (i, 0)),
        out_shape=jax.ShapeDtypeStruct((N_NODES, D), jnp.float32),
    )(x, W0, b0r)


def _tc_dinv(degp):
    def body(p_ref, o_ref):
        deg = p_ref[0] + p_ref[1] + 1.0
        o_ref[...] = lax.rsqrt(deg)

    return pl.pallas_call(
        body,
        in_specs=[pl.BlockSpec((NC, N_PAD // 128, 128), lambda: (0, 0, 0))],
        out_specs=pl.BlockSpec((N_PAD // 128, 128), lambda: (0, 0)),
        out_shape=jax.ShapeDtypeStruct((N_PAD // 128, 128), jnp.float32),
    )(degp)


def _tc_scale_mm(h, W, dinv_col):
    def body(h_ref, w_ref, d_ref, o_ref):
        y = jnp.dot(h_ref[...], w_ref[...], preferred_element_type=jnp.float32)
        o_ref[...] = y * d_ref[...]

    return pl.pallas_call(
        body,
        grid=(_GRID,),
        in_specs=[
            pl.BlockSpec((_R, D), lambda i: (i, 0)),
            pl.BlockSpec((D, D), lambda i: (0, 0)),
            pl.BlockSpec((_R, 1), lambda i: (i, 0)),
        ],
        out_specs=pl.BlockSpec((_R, D), lambda i: (i, 0)),
        out_shape=jax.ShapeDtypeStruct((N_NODES, D), jnp.float32),
    )(h, W, dinv_col)


def _tc_mid(p, y, dinv_col, br, W):
    def body(p_ref, y_ref, d_ref, b_ref, w_ref, o_ref):
        agg = p_ref[0] + p_ref[1] + y_ref[...]
        h = jnp.maximum(agg * d_ref[...] + b_ref[...], 0.0)
        o_ref[...] = jnp.dot(h, w_ref[...],
                             preferred_element_type=jnp.float32) * d_ref[...]

    return pl.pallas_call(
        body,
        grid=(_GRID,),
        in_specs=[
            pl.BlockSpec((NC, _R, D), lambda i: (0, i, 0)),
            pl.BlockSpec((_R, D), lambda i: (i, 0)),
            pl.BlockSpec((_R, 1), lambda i: (i, 0)),
            pl.BlockSpec((1, D), lambda i: (0, 0)),
            pl.BlockSpec((D, D), lambda i: (0, 0)),
        ],
        out_specs=pl.BlockSpec((_R, D), lambda i: (i, 0)),
        out_shape=jax.ShapeDtypeStruct((N_NODES, D), jnp.float32),
    )(p, y, dinv_col, br, W)


def _tc_out(p, y, dinv_col, br, W, b3r):
    def body(p_ref, y_ref, d_ref, b_ref, w_ref, b3_ref, o_ref):
        agg = p_ref[0] + p_ref[1] + y_ref[...]
        h = jnp.maximum(agg * d_ref[...] + b_ref[...], 0.0)
        o_ref[...] = jnp.dot(h, w_ref[...],
                             preferred_element_type=jnp.float32) + b3_ref[...]

    return pl.pallas_call(
        body,
        grid=(_GRID,),
        in_specs=[
            pl.BlockSpec((NC, _R, D), lambda i: (0, i, 0)),
            pl.BlockSpec((_R, D), lambda i: (i, 0)),
            pl.BlockSpec((_R, 1), lambda i: (i, 0)),
            pl.BlockSpec((1, D), lambda i: (0, 0)),
            pl.BlockSpec((D, D), lambda i: (0, 0)),
            pl.BlockSpec((1, D), lambda i: (0, 0)),
        ],
        out_specs=pl.BlockSpec((_R, D), lambda i: (i, 0)),
        out_shape=jax.ShapeDtypeStruct((N_NODES, D), jnp.float32),
    )(p, y, dinv_col, br, W, b3r)


def kernel(x, edge_index, W0, b0, W1, b1, W2, b2, W3, b3):
    ei = edge_index.astype(jnp.int32)
    pad = E_PAD - E
    src3 = jnp.concatenate(
        [ei[0], jnp.zeros((pad,), jnp.int32)]).reshape(NW, NCH, CHUNK)
    pad_dst = N_NODES + jnp.arange(pad, dtype=jnp.int32) % (N_PAD - N_NODES)
    dst_pad = jnp.concatenate([ei[1], pad_dst])
    dst3 = dst_pad.reshape(NW, NCH, CHUNK)

    zeros1 = jnp.zeros((N_PAD,), jnp.float32)
    zeros2 = jnp.zeros((N_PAD, D), jnp.float32)
    b0r = b0.reshape(1, D)
    b1r = b1.reshape(1, D)
    b2r = b2.reshape(1, D)
    b3r = b3.reshape(1, D)

    _sc_deg, _sc_msg = _sc_kernels()
    degp = _sc_deg(dst3, zeros1)
    h0 = _tc_in(x, W0, b0r)
    dinv2d = _tc_dinv(degp.reshape(NC, N_PAD // 128, 128))
    dinv_col = dinv2d.reshape(N_PAD, 1)[:N_NODES]

    y1 = _tc_scale_mm(h0, W1, dinv_col)
    p1 = _sc_msg(src3, dst3, y1, zeros2)
    y2 = _tc_mid(p1, y1, dinv_col, b1r, W2)
    p2 = _sc_msg(src3, dst3, y2, zeros2)
    logits = _tc_out(p2, y2, dinv_col, b2r, W3, b3r)
    return logits


# speedup vs baseline: 1.0951x; 1.0951x over previous
import functools

import jax
import jax.numpy as jnp
from jax import lax
from jax.experimental import pallas as pl
from jax.experimental.pallas import tpu as pltpu
from jax.experimental.pallas import tpu_sc as plsc

N_NODES = 10000
D = 128
E = 320000

NC = 2
NS = 16
NW = NC * NS
CHUNK = 128
EPW = E // NW
NCH = 80
EPW_PAD = NCH * CHUNK
E_PAD = EPW_PAD * NW
NCHH = NCH // 2
GRP = 8
N_PAD = 10240
DST_PAD = 10008
RPT = N_PAD // NS

@functools.cache
def _sc_kernels():
    mesh = plsc.VectorSubcoreMesh(core_axis_name="c", subcore_axis_name="s",
                                  num_cores=NC, num_subcores=NS)

    @functools.partial(
        pl.kernel,
        out_type=jax.ShapeDtypeStruct((NC, N_PAD), jnp.float32),
        mesh=mesh,
        scratch_types=[
            pltpu.VMEM((NCH, CHUNK), jnp.int32),
            pltpu.VMEM((CHUNK,), jnp.float32),
            pltpu.VMEM_SHARED((N_PAD,), jnp.float32),
        ],
    )
    def sc_deg(dst_hbm, zeros1_hbm, out_hbm, dst_v, ones_v, deg_sh):
        c = lax.axis_index("c")
        s = lax.axis_index("s")
        wid = s * NC + c
        for i in range(CHUNK // 16):
            ones_v[pl.ds(i * 16, 16)] = jnp.ones((16,), jnp.float32)
        pltpu.sync_copy(zeros1_hbm.at[pl.ds(s * RPT, RPT)],
                        deg_sh.at[pl.ds(s * RPT, RPT)])
        plsc.subcore_barrier()
        pltpu.sync_copy(dst_hbm.at[wid], dst_v)

        def body(j, carry):
            pltpu.sync_copy(ones_v, deg_sh.at[dst_v.at[j]], add=True)
            return carry

        lax.fori_loop(0, NCH, body, 0)
        plsc.subcore_barrier()
        pltpu.sync_copy(deg_sh.at[pl.ds(s * RPT, RPT)],
                        out_hbm.at[c, pl.ds(s * RPT, RPT)])

    @functools.partial(
        pl.kernel,
        out_type=jax.ShapeDtypeStruct((NC, N_PAD, D), jnp.float32),
        mesh=mesh,
        scratch_types=[
            pltpu.VMEM((NCHH, CHUNK), jnp.int32),
            pltpu.VMEM((NCHH, CHUNK), jnp.int32),
            [pltpu.VMEM((CHUNK, D), jnp.float32) for _ in range(2)],
            pltpu.VMEM_SHARED((N_PAD, D), jnp.float32),
            [pltpu.SemaphoreType.DMA for _ in range(2)],
        ],
    )
    def sc_msg(src_hbm, dst_hbm, y_hbm, zeros2_hbm, out_hbm,
               src_v, dst_v, rows, agg_sh, gsem):
        c = lax.axis_index("c")
        s = lax.axis_index("s")
        wid = s * NC + c
        pltpu.sync_copy(zeros2_hbm.at[pl.ds(s * RPT, RPT)],
                        agg_sh.at[pl.ds(s * RPT, RPT)])
        plsc.subcore_barrier()

        def run_group(j0, n):
            descs = [None, None]
            descs[0] = pltpu.async_copy(
                y_hbm.at[src_v.at[j0]], rows[0], gsem[0])
            for k in range(n):
                b = k % 2
                if k + 1 < n:
                    nb = 1 - b
                    descs[nb] = pltpu.async_copy(
                        y_hbm.at[src_v.at[j0 + k + 1]], rows[nb], gsem[nb])
                descs[b].wait()
                pltpu.sync_copy(rows[b], agg_sh.at[dst_v.at[j0 + k]],
                                add=True)

        for half, nch in ((0, NCHH), (1, NCHH - 1)):
            pltpu.sync_copy(src_hbm.at[wid, pl.ds(half * NCHH, NCHH)],
                            src_v)
            pltpu.sync_copy(dst_hbm.at[wid, pl.ds(half * NCHH, NCHH)],
                            dst_v)
            ngrp = nch // GRP

            def body(grp, carry):
                run_group(grp * GRP, GRP)
                return carry

            lax.fori_loop(0, ngrp, body, 0)
            if nch % GRP:
                run_group(ngrp * GRP, nch % GRP)
        plsc.subcore_barrier()
        pltpu.sync_copy(agg_sh.at[pl.ds(s * RPT, RPT)],
                        out_hbm.at[c, pl.ds(s * RPT, RPT)])

    return sc_deg, sc_msg


_R = 2000
_GRID = N_NODES // _R


def _tc_in(x, W0, b0r):
    def body(x_ref, w_ref, b_ref, o_ref):
        h = jnp.dot(x_ref[...], w_ref[...], preferred_element_type=jnp.float32)
        o_ref[...] = jnp.maximum(h + b_ref[...], 0.0)

    return pl.pallas_call(
        body,
        grid=(_GRID,),
        in_specs=[
            pl.BlockSpec((_R, D), lambda i: (i, 0)),
            pl.BlockSpec((D, D), lambda i: (0, 0)),
            pl.BlockSpec((1, D), lambda i: (0, 0)),
        ],
        out_specs=pl.BlockSpec((_R, D), lambda i: (i, 0)),
        out_shape=jax.ShapeDtypeStruct((N_NODES, D), jnp.float32),
    )(x, W0, b0r)


def _tc_dinv(degp):
    def body(p_ref, o_ref):
        deg = p_ref[0] + p_ref[1] + 1.0
        o_ref[...] = lax.rsqrt(deg)

    return pl.pallas_call(
        body,
        in_specs=[pl.BlockSpec((NC, N_PAD // 128, 128), lambda: (0, 0, 0))],
        out_specs=pl.BlockSpec((N_PAD // 128, 128), lambda: (0, 0)),
        out_shape=jax.ShapeDtypeStruct((N_PAD // 128, 128), jnp.float32),
    )(degp)


def _tc_scale_mm(h, W, dinv_col):
    def body(h_ref, w_ref, d_ref, o_ref):
        y = jnp.dot(h_ref[...], w_ref[...], preferred_element_type=jnp.float32)
        o_ref[...] = y * d_ref[...]

    return pl.pallas_call(
        body,
        grid=(_GRID,),
        in_specs=[
            pl.BlockSpec((_R, D), lambda i: (i, 0)),
            pl.BlockSpec((D, D), lambda i: (0, 0)),
            pl.BlockSpec((_R, 1), lambda i: (i, 0)),
        ],
        out_specs=pl.BlockSpec((_R, D), lambda i: (i, 0)),
        out_shape=jax.ShapeDtypeStruct((N_NODES, D), jnp.float32),
    )(h, W, dinv_col)


def _tc_mid(p, y, dinv_col, br, W):
    def body(p_ref, y_ref, d_ref, b_ref, w_ref, o_ref):
        agg = p_ref[0] + p_ref[1] + y_ref[...]
        h = jnp.maximum(agg * d_ref[...] + b_ref[...], 0.0)
        o_ref[...] = jnp.dot(h, w_ref[...],
                             preferred_element_type=jnp.float32) * d_ref[...]

    return pl.pallas_call(
        body,
        grid=(_GRID,),
        in_specs=[
            pl.BlockSpec((NC, _R, D), lambda i: (0, i, 0)),
            pl.BlockSpec((_R, D), lambda i: (i, 0)),
            pl.BlockSpec((_R, 1), lambda i: (i, 0)),
            pl.BlockSpec((1, D), lambda i: (0, 0)),
            pl.BlockSpec((D, D), lambda i: (0, 0)),
        ],
        out_specs=pl.BlockSpec((_R, D), lambda i: (i, 0)),
        out_shape=jax.ShapeDtypeStruct((N_NODES, D), jnp.float32),
    )(p, y, dinv_col, br, W)


def _tc_out(p, y, dinv_col, br, W, b3r):
    def body(p_ref, y_ref, d_ref, b_ref, w_ref, b3_ref, o_ref):
        agg = p_ref[0] + p_ref[1] + y_ref[...]
        h = jnp.maximum(agg * d_ref[...] + b_ref[...], 0.0)
        o_ref[...] = jnp.dot(h, w_ref[...],
                             preferred_element_type=jnp.float32) + b3_ref[...]

    return pl.pallas_call(
        body,
        grid=(_GRID,),
        in_specs=[
            pl.BlockSpec((NC, _R, D), lambda i: (0, i, 0)),
            pl.BlockSpec((_R, D), lambda i: (i, 0)),
            pl.BlockSpec((_R, 1), lambda i: (i, 0)),
            pl.BlockSpec((1, D), lambda i: (0, 0)),
            pl.BlockSpec((D, D), lambda i: (0, 0)),
            pl.BlockSpec((1, D), lambda i: (0, 0)),
        ],
        out_specs=pl.BlockSpec((_R, D), lambda i: (i, 0)),
        out_shape=jax.ShapeDtypeStruct((N_NODES, D), jnp.float32),
    )(p, y, dinv_col, br, W, b3r)


def kernel(x, edge_index, W0, b0, W1, b1, W2, b2, W3, b3):
    ei = edge_index.astype(jnp.int32)
    pad = E_PAD - E
    src3 = jnp.concatenate(
        [ei[0], jnp.zeros((pad,), jnp.int32)]).reshape(NW, NCH, CHUNK)
    pad_dst = N_NODES + jnp.arange(pad, dtype=jnp.int32) % (N_PAD - N_NODES)
    dst_pad = jnp.concatenate([ei[1], pad_dst])
    dst3 = dst_pad.reshape(NW, NCH, CHUNK)

    zeros1 = jnp.zeros((N_PAD,), jnp.float32)
    zeros2 = jnp.zeros((N_PAD, D), jnp.float32)
    b0r = b0.reshape(1, D)
    b1r = b1.reshape(1, D)
    b2r = b2.reshape(1, D)
    b3r = b3.reshape(1, D)

    _sc_deg, _sc_msg = _sc_kernels()
    degp = _sc_deg(dst3, zeros1)
    h0 = _tc_in(x, W0, b0r)
    dinv2d = _tc_dinv(degp.reshape(NC, N_PAD // 128, 128))
    dinv_col = dinv2d.reshape(N_PAD, 1)[:N_NODES]

    y1 = _tc_scale_mm(h0, W1, dinv_col)
    p1 = _sc_msg(src3, dst3, y1, zeros2)
    y2 = _tc_mid(p1, y1, dinv_col, b1r, W2)
    p2 = _sc_msg(src3, dst3, y2, zeros2)
    logits = _tc_out(p2, y2, dinv_col, b2r, W3, b3r)
    return logits


# speedup vs baseline: 1.1567x; 1.0562x over previous
import functools

import jax
import jax.numpy as jnp
from jax import lax
from jax.experimental import pallas as pl
from jax.experimental.pallas import tpu as pltpu
from jax.experimental.pallas import tpu_sc as plsc

N_NODES = 10000
D = 128
E = 320000

NC = 2
NS = 16
NW = NC * NS
CHUNK = 128
EPW = E // NW
NCH = 80
EPW_PAD = NCH * CHUNK
E_PAD = EPW_PAD * NW
NCHH = NCH // 2
GRP = 8
N_PAD = 10240
DST_PAD = 10008
RPT = N_PAD // NS

@functools.cache
def _sc_kernels():
    mesh = plsc.VectorSubcoreMesh(core_axis_name="c", subcore_axis_name="s",
                                  num_cores=NC, num_subcores=NS)

    @functools.partial(
        pl.kernel,
        out_type=jax.ShapeDtypeStruct((NC, N_PAD), jnp.float32),
        mesh=mesh,
        scratch_types=[
            pltpu.VMEM((NCH, CHUNK), jnp.int32),
            pltpu.VMEM((CHUNK,), jnp.float32),
            pltpu.VMEM_SHARED((N_PAD,), jnp.float32),
        ],
    )
    def sc_deg(dst_hbm, zeros1_hbm, out_hbm, dst_v, ones_v, deg_sh):
        c = lax.axis_index("c")
        s = lax.axis_index("s")
        wid = s * NC + c
        for i in range(CHUNK // 16):
            ones_v[pl.ds(i * 16, 16)] = jnp.ones((16,), jnp.float32)
        pltpu.sync_copy(zeros1_hbm.at[pl.ds(s * RPT, RPT)],
                        deg_sh.at[pl.ds(s * RPT, RPT)])
        plsc.subcore_barrier()
        pltpu.sync_copy(dst_hbm.at[wid], dst_v)

        def body(j, carry):
            pltpu.sync_copy(ones_v, deg_sh.at[dst_v.at[j]], add=True)
            return carry

        lax.fori_loop(0, NCH, body, 0)
        plsc.subcore_barrier()
        pltpu.sync_copy(deg_sh.at[pl.ds(s * RPT, RPT)],
                        out_hbm.at[c, pl.ds(s * RPT, RPT)])

    @functools.partial(
        pl.kernel,
        out_type=jax.ShapeDtypeStruct((NC, N_PAD, D), jnp.float32),
        mesh=mesh,
        scratch_types=[
            pltpu.VMEM((NCH, CHUNK), jnp.int32),
            pltpu.VMEM((NCH, CHUNK), jnp.int32),
            pltpu.VMEM((CHUNK, D), jnp.float32),
            pltpu.VMEM_SHARED((N_PAD, D), jnp.float32),
            pltpu.SemaphoreType.DMA,
        ],
    )
    def sc_msg(src_hbm, dst_hbm, y_hbm, zeros2_hbm, out_hbm,
               src_v, dst_v, rows_v, agg_sh, gsem):
        c = lax.axis_index("c")
        s = lax.axis_index("s")
        wid = s * NC + c
        pltpu.sync_copy(zeros2_hbm.at[pl.ds(s * RPT, RPT)],
                        agg_sh.at[pl.ds(s * RPT, RPT)])
        pltpu.sync_copy(src_hbm.at[wid], src_v)
        pltpu.sync_copy(dst_hbm.at[wid], dst_v)
        plsc.subcore_barrier()

        def body(j, carry):
            pltpu.async_copy(y_hbm.at[src_v.at[j]], rows_v, gsem).wait()
            pltpu.sync_copy(rows_v, agg_sh.at[dst_v.at[j]], add=True)
            return carry

        lax.fori_loop(0, NCH, body, 0)
        plsc.subcore_barrier()
        pltpu.sync_copy(agg_sh.at[pl.ds(s * RPT, RPT)],
                        out_hbm.at[c, pl.ds(s * RPT, RPT)])

    return sc_deg, sc_msg


_R = 2000
_GRID = N_NODES // _R


def _tc_in(x, W0, b0r):
    def body(x_ref, w_ref, b_ref, o_ref):
        h = jnp.dot(x_ref[...], w_ref[...], preferred_element_type=jnp.float32)
        o_ref[...] = jnp.maximum(h + b_ref[...], 0.0)

    return pl.pallas_call(
        body,
        grid=(_GRID,),
        in_specs=[
            pl.BlockSpec((_R, D), lambda i: (i, 0)),
            pl.BlockSpec((D, D), lambda i: (0, 0)),
            pl.BlockSpec((1, D), lambda i: (0, 0)),
        ],
        out_specs=pl.BlockSpec((_R, D), lambda i: (i, 0)),
        out_shape=jax.ShapeDtypeStruct((N_NODES, D), jnp.float32),
    )(x, W0, b0r)


def _tc_dinv(degp):
    def body(p_ref, o_ref):
        deg = p_ref[0] + p_ref[1] + 1.0
        o_ref[...] = lax.rsqrt(deg)

    return pl.pallas_call(
        body,
        in_specs=[pl.BlockSpec((NC, N_PAD // 128, 128), lambda: (0, 0, 0))],
        out_specs=pl.BlockSpec((N_PAD // 128, 128), lambda: (0, 0)),
        out_shape=jax.ShapeDtypeStruct((N_PAD // 128, 128), jnp.float32),
    )(degp)


def _tc_scale_mm(h, W, dinv_col):
    def body(h_ref, w_ref, d_ref, o_ref):
        y = jnp.dot(h_ref[...], w_ref[...], preferred_element_type=jnp.float32)
        o_ref[...] = y * d_ref[...]

    return pl.pallas_call(
        body,
        grid=(_GRID,),
        in_specs=[
            pl.BlockSpec((_R, D), lambda i: (i, 0)),
            pl.BlockSpec((D, D), lambda i: (0, 0)),
            pl.BlockSpec((_R, 1), lambda i: (i, 0)),
        ],
        out_specs=pl.BlockSpec((_R, D), lambda i: (i, 0)),
        out_shape=jax.ShapeDtypeStruct((N_NODES, D), jnp.float32),
    )(h, W, dinv_col)


def _tc_mid(p, y, dinv_col, br, W):
    def body(p_ref, y_ref, d_ref, b_ref, w_ref, o_ref):
        agg = p_ref[0] + p_ref[1] + y_ref[...]
        h = jnp.maximum(agg * d_ref[...] + b_ref[...], 0.0)
        o_ref[...] = jnp.dot(h, w_ref[...],
                             preferred_element_type=jnp.float32) * d_ref[...]

    return pl.pallas_call(
        body,
        grid=(_GRID,),
        in_specs=[
            pl.BlockSpec((NC, _R, D), lambda i: (0, i, 0)),
            pl.BlockSpec((_R, D), lambda i: (i, 0)),
            pl.BlockSpec((_R, 1), lambda i: (i, 0)),
            pl.BlockSpec((1, D), lambda i: (0, 0)),
            pl.BlockSpec((D, D), lambda i: (0, 0)),
        ],
        out_specs=pl.BlockSpec((_R, D), lambda i: (i, 0)),
        out_shape=jax.ShapeDtypeStruct((N_NODES, D), jnp.float32),
    )(p, y, dinv_col, br, W)


def _tc_out(p, y, dinv_col, br, W, b3r):
    def body(p_ref, y_ref, d_ref, b_ref, w_ref, b3_ref, o_ref):
        agg = p_ref[0] + p_ref[1] + y_ref[...]
        h = jnp.maximum(agg * d_ref[...] + b_ref[...], 0.0)
        o_ref[...] = jnp.dot(h, w_ref[...],
                             preferred_element_type=jnp.float32) + b3_ref[...]

    return pl.pallas_call(
        body,
        grid=(_GRID,),
        in_specs=[
            pl.BlockSpec((NC, _R, D), lambda i: (0, i, 0)),
            pl.BlockSpec((_R, D), lambda i: (i, 0)),
            pl.BlockSpec((_R, 1), lambda i: (i, 0)),
            pl.BlockSpec((1, D), lambda i: (0, 0)),
            pl.BlockSpec((D, D), lambda i: (0, 0)),
            pl.BlockSpec((1, D), lambda i: (0, 0)),
        ],
        out_specs=pl.BlockSpec((_R, D), lambda i: (i, 0)),
        out_shape=jax.ShapeDtypeStruct((N_NODES, D), jnp.float32),
    )(p, y, dinv_col, br, W, b3r)


def kernel(x, edge_index, W0, b0, W1, b1, W2, b2, W3, b3):
    ei = edge_index.astype(jnp.int32)
    pad = E_PAD - E
    src3 = jnp.concatenate(
        [ei[0], jnp.zeros((pad,), jnp.int32)]
    ).reshape(NCH, NW, CHUNK).transpose(1, 0, 2)
    pad_dst = N_NODES + jnp.arange(pad, dtype=jnp.int32) % (N_PAD - N_NODES)
    dst3 = jnp.concatenate(
        [ei[1], pad_dst]).reshape(NCH, NW, CHUNK).transpose(1, 0, 2)

    zeros1 = jnp.zeros((N_PAD,), jnp.float32)
    zeros2 = jnp.zeros((N_PAD, D), jnp.float32)
    b0r = b0.reshape(1, D)
    b1r = b1.reshape(1, D)
    b2r = b2.reshape(1, D)
    b3r = b3.reshape(1, D)

    _sc_deg, _sc_msg = _sc_kernels()
    degp = _sc_deg(dst3, zeros1)
    h0 = _tc_in(x, W0, b0r)
    dinv2d = _tc_dinv(degp.reshape(NC, N_PAD // 128, 128))
    dinv_col = dinv2d.reshape(N_PAD, 1)[:N_NODES]

    y1 = _tc_scale_mm(h0, W1, dinv_col)
    p1 = _sc_msg(src3, dst3, y1, zeros2)
    y2 = _tc_mid(p1, y1, dinv_col, b1r, W2)
    p2 = _sc_msg(src3, dst3, y2, zeros2)
    logits = _tc_out(p2, y2, dinv_col, b2r, W3, b3r)
    return logits


# speedup vs baseline: 2.6440x; 2.2858x over previous
import functools

import jax
import jax.numpy as jnp
from jax import lax
from jax.experimental import pallas as pl
from jax.experimental.pallas import tpu as pltpu
from jax.experimental.pallas import tpu_sc as plsc

N_NODES = 10000
D = 128
E = 320000

NC = 2
NS = 16
NW = NC * NS
CHUNK = 128
EPW = E // NW
NCH = 80
EPW_PAD = NCH * CHUNK
E_PAD = EPW_PAD * NW
NCHH = NCH // 2
GRP = 8
N_PAD = 10240
DST_PAD = 10008
RPT = N_PAD // NS

@functools.cache
def _sc_kernels():
    mesh = plsc.VectorSubcoreMesh(core_axis_name="c", subcore_axis_name="s",
                                  num_cores=NC, num_subcores=NS)

    @functools.partial(
        pl.kernel,
        out_type=jax.ShapeDtypeStruct((NC, N_PAD), jnp.float32),
        mesh=mesh,
        scratch_types=[
            pltpu.VMEM((NCH, CHUNK), jnp.int32),
            pltpu.VMEM((CHUNK,), jnp.float32),
            pltpu.VMEM_SHARED((N_PAD,), jnp.float32),
        ],
    )
    def sc_deg(dst_hbm, zeros1_hbm, out_hbm, dst_v, ones_v, deg_sh):
        c = lax.axis_index("c")
        s = lax.axis_index("s")
        wid = s * NC + c
        for i in range(CHUNK // 16):
            ones_v[pl.ds(i * 16, 16)] = jnp.ones((16,), jnp.float32)
        pltpu.sync_copy(zeros1_hbm.at[pl.ds(s * RPT, RPT)],
                        deg_sh.at[pl.ds(s * RPT, RPT)])
        plsc.subcore_barrier()
        pltpu.sync_copy(dst_hbm.at[wid], dst_v)

        def body(j, carry):
            pltpu.sync_copy(ones_v, deg_sh.at[dst_v.at[j]], add=True)
            return carry

        lax.fori_loop(0, NCH, body, 0)
        plsc.subcore_barrier()
        pltpu.sync_copy(deg_sh.at[pl.ds(s * RPT, RPT)],
                        out_hbm.at[c, pl.ds(s * RPT, RPT)])

    @functools.partial(
        pl.kernel,
        out_type=jax.ShapeDtypeStruct((NC, N_PAD, D), jnp.float32),
        mesh=mesh,
        scratch_types=[
            pltpu.VMEM((NCH, CHUNK), jnp.int32),
            pltpu.VMEM((NCH, CHUNK), jnp.int32),
            pltpu.VMEM((CHUNK, D), jnp.float32),
            pltpu.VMEM_SHARED((N_PAD, D), jnp.float32),
            pltpu.SemaphoreType.DMA,
        ],
    )
    def sc_msg(src_hbm, dst_hbm, y_hbm, zeros2_hbm, out_hbm,
               src_v, dst_v, rows_v, agg_sh, gsem):
        c = lax.axis_index("c")
        s = lax.axis_index("s")
        wid = s * NC + c
        pltpu.sync_copy(zeros2_hbm.at[pl.ds(s * RPT, RPT)],
                        agg_sh.at[pl.ds(s * RPT, RPT)])
        pltpu.sync_copy(src_hbm.at[wid], src_v)
        pltpu.sync_copy(dst_hbm.at[wid], dst_v)
        plsc.subcore_barrier()

        def body(j, carry):
            pltpu.async_copy(y_hbm.at[src_v.at[j]], rows_v, gsem).wait()
            pltpu.sync_copy(rows_v, agg_sh.at[dst_v.at[j]], add=True)
            return carry

        lax.fori_loop(0, NCH, body, 0)
        plsc.subcore_barrier()
        pltpu.sync_copy(agg_sh.at[pl.ds(s * RPT, RPT)],
                        out_hbm.at[c, pl.ds(s * RPT, RPT)])

    return sc_deg, sc_msg


_R = 2000
_GRID = N_NODES // _R


def _tc_in(x, W0, b0r):
    def body(x_ref, w_ref, b_ref, o_ref):
        h = jnp.dot(x_ref[...], w_ref[...], preferred_element_type=jnp.float32)
        o_ref[...] = jnp.maximum(h + b_ref[...], 0.0)

    return pl.pallas_call(
        body,
        grid=(_GRID,),
        in_specs=[
            pl.BlockSpec((_R, D), lambda i: (i, 0)),
            pl.BlockSpec((D, D), lambda i: (0, 0)),
            pl.BlockSpec((1, D), lambda i: (0, 0)),
        ],
        out_specs=pl.BlockSpec((_R, D), lambda i: (i, 0)),
        out_shape=jax.ShapeDtypeStruct((N_NODES, D), jnp.float32),
    )(x, W0, b0r)


def _tc_dinv(degp):
    def body(p_ref, o_ref):
        deg = p_ref[0] + p_ref[1] + 1.0
        o_ref[...] = lax.rsqrt(deg)

    return pl.pallas_call(
        body,
        in_specs=[pl.BlockSpec((NC, N_PAD // 128, 128), lambda: (0, 0, 0))],
        out_specs=pl.BlockSpec((N_PAD // 128, 128), lambda: (0, 0)),
        out_shape=jax.ShapeDtypeStruct((N_PAD // 128, 128), jnp.float32),
    )(degp)


def _tc_scale_mm(h, W, dinv_col):
    def body(h_ref, w_ref, d_ref, o_ref):
        y = jnp.dot(h_ref[...], w_ref[...], preferred_element_type=jnp.float32)
        o_ref[...] = y * d_ref[...]

    return pl.pallas_call(
        body,
        grid=(_GRID,),
        in_specs=[
            pl.BlockSpec((_R, D), lambda i: (i, 0)),
            pl.BlockSpec((D, D), lambda i: (0, 0)),
            pl.BlockSpec((_R, 1), lambda i: (i, 0)),
        ],
        out_specs=pl.BlockSpec((_R, D), lambda i: (i, 0)),
        out_shape=jax.ShapeDtypeStruct((N_NODES, D), jnp.float32),
    )(h, W, dinv_col)


def _tc_mid(p, y, dinv_col, br, W):
    def body(p_ref, y_ref, d_ref, b_ref, w_ref, o_ref):
        agg = p_ref[0] + p_ref[1] + y_ref[...]
        h = jnp.maximum(agg * d_ref[...] + b_ref[...], 0.0)
        o_ref[...] = jnp.dot(h, w_ref[...],
                             preferred_element_type=jnp.float32) * d_ref[...]

    return pl.pallas_call(
        body,
        grid=(_GRID,),
        in_specs=[
            pl.BlockSpec((NC, _R, D), lambda i: (0, i, 0)),
            pl.BlockSpec((_R, D), lambda i: (i, 0)),
            pl.BlockSpec((_R, 1), lambda i: (i, 0)),
            pl.BlockSpec((1, D), lambda i: (0, 0)),
            pl.BlockSpec((D, D), lambda i: (0, 0)),
        ],
        out_specs=pl.BlockSpec((_R, D), lambda i: (i, 0)),
        out_shape=jax.ShapeDtypeStruct((N_NODES, D), jnp.float32),
    )(p, y, dinv_col, br, W)


def _tc_out(p, y, dinv_col, br, W, b3r):
    def body(p_ref, y_ref, d_ref, b_ref, w_ref, b3_ref, o_ref):
        agg = p_ref[0] + p_ref[1] + y_ref[...]
        h = jnp.maximum(agg * d_ref[...] + b_ref[...], 0.0)
        o_ref[...] = jnp.dot(h, w_ref[...],
                             preferred_element_type=jnp.float32) + b3_ref[...]

    return pl.pallas_call(
        body,
        grid=(_GRID,),
        in_specs=[
            pl.BlockSpec((NC, _R, D), lambda i: (0, i, 0)),
            pl.BlockSpec((_R, D), lambda i: (i, 0)),
            pl.BlockSpec((_R, 1), lambda i: (i, 0)),
            pl.BlockSpec((1, D), lambda i: (0, 0)),
            pl.BlockSpec((D, D), lambda i: (0, 0)),
            pl.BlockSpec((1, D), lambda i: (0, 0)),
        ],
        out_specs=pl.BlockSpec((_R, D), lambda i: (i, 0)),
        out_shape=jax.ShapeDtypeStruct((N_NODES, D), jnp.float32),
    )(p, y, dinv_col, br, W, b3r)


def kernel(x, edge_index, W0, b0, W1, b1, W2, b2, W3, b3):
    ei = edge_index.astype(jnp.int32)
    pad = E_PAD - E
    pad_src = jnp.arange(pad, dtype=jnp.int32) % N_NODES
    src3 = jnp.concatenate(
        [ei[0], pad_src]).reshape(NCH, NW, CHUNK).transpose(1, 0, 2)
    pad_dst = N_NODES + jnp.arange(pad, dtype=jnp.int32) % (N_PAD - N_NODES)
    dst3 = jnp.concatenate(
        [ei[1], pad_dst]).reshape(NCH, NW, CHUNK).transpose(1, 0, 2)

    zeros1 = jnp.zeros((N_PAD,), jnp.float32)
    zeros2 = jnp.zeros((N_PAD, D), jnp.float32)
    b0r = b0.reshape(1, D)
    b1r = b1.reshape(1, D)
    b2r = b2.reshape(1, D)
    b3r = b3.reshape(1, D)

    _sc_deg, _sc_msg = _sc_kernels()
    degp = _sc_deg(dst3, zeros1)
    h0 = _tc_in(x, W0, b0r)
    dinv2d = _tc_dinv(degp.reshape(NC, N_PAD // 128, 128))
    dinv_col = dinv2d.reshape(N_PAD, 1)[:N_NODES]

    y1 = _tc_scale_mm(h0, W1, dinv_col)
    p1 = _sc_msg(src3, dst3, y1, zeros2)
    y2 = _tc_mid(p1, y1, dinv_col, b1r, W2)
    p2 = _sc_msg(src3, dst3, y2, zeros2)
    logits = _tc_out(p2, y2, dinv_col, b2r, W3, b3r)
    return logits


# speedup vs baseline: 3.5465x; 1.3413x over previous
import functools

import jax
import jax.numpy as jnp
from jax import lax
from jax.experimental import pallas as pl
from jax.experimental.pallas import tpu as pltpu
from jax.experimental.pallas import tpu_sc as plsc

N_NODES = 10000
D = 128
E = 320000

NC = 2
NS = 16
NW = NC * NS
CHUNK = 128
EPW = E // NW
NCH = 80
EPW_PAD = NCH * CHUNK
E_PAD = EPW_PAD * NW
NCHH = NCH // 2
GRP = 8
N_PAD = 10240
DST_PAD = 10008
RPT = N_PAD // NS

@functools.cache
def _sc_kernels():
    mesh = plsc.VectorSubcoreMesh(core_axis_name="c", subcore_axis_name="s",
                                  num_cores=NC, num_subcores=NS)

    @functools.partial(
        pl.kernel,
        out_type=jax.ShapeDtypeStruct((NC, N_PAD), jnp.float32),
        mesh=mesh,
        scratch_types=[
            pltpu.VMEM((NCH, CHUNK), jnp.int32),
            pltpu.VMEM((CHUNK,), jnp.float32),
            pltpu.VMEM_SHARED((N_PAD,), jnp.float32),
        ],
    )
    def sc_deg(dst_hbm, zeros1_hbm, out_hbm, dst_v, ones_v, deg_sh):
        c = lax.axis_index("c")
        s = lax.axis_index("s")
        wid = s * NC + c
        for i in range(CHUNK // 16):
            ones_v[pl.ds(i * 16, 16)] = jnp.ones((16,), jnp.float32)
        pltpu.sync_copy(zeros1_hbm.at[pl.ds(s * RPT, RPT)],
                        deg_sh.at[pl.ds(s * RPT, RPT)])
        plsc.subcore_barrier()
        pltpu.sync_copy(dst_hbm.at[wid], dst_v)

        def body(j, carry):
            pltpu.sync_copy(ones_v, deg_sh.at[dst_v.at[j]], add=True)
            return carry

        lax.fori_loop(0, NCH, body, 0)
        plsc.subcore_barrier()
        pltpu.sync_copy(deg_sh.at[pl.ds(s * RPT, RPT)],
                        out_hbm.at[c, pl.ds(s * RPT, RPT)])

    @functools.partial(
        pl.kernel,
        out_type=jax.ShapeDtypeStruct((NC, N_PAD, D), jnp.float32),
        mesh=mesh,
        scratch_types=[
            pltpu.VMEM((NCHH, CHUNK), jnp.int32),
            pltpu.VMEM((NCHH, CHUNK), jnp.int32),
            [pltpu.VMEM((CHUNK, D), jnp.float32) for _ in range(2)],
            pltpu.VMEM_SHARED((N_PAD, D), jnp.float32),
            [pltpu.SemaphoreType.DMA for _ in range(2)],
        ],
    )
    def sc_msg(src_hbm, dst_hbm, y_hbm, zeros2_hbm, out_hbm,
               src_v, dst_v, rows, agg_sh, gsem):
        c = lax.axis_index("c")
        s = lax.axis_index("s")
        wid = s * NC + c
        pltpu.sync_copy(zeros2_hbm.at[pl.ds(s * RPT, RPT)],
                        agg_sh.at[pl.ds(s * RPT, RPT)])
        plsc.subcore_barrier()
        for half in range(2):
            pltpu.sync_copy(src_hbm.at[wid, pl.ds(half * NCHH, NCHH)],
                            src_v)
            pltpu.sync_copy(dst_hbm.at[wid, pl.ds(half * NCHH, NCHH)],
                            dst_v)

            def body(grp, carry):
                j0 = grp * GRP
                descs = [None, None]
                descs[0] = pltpu.async_copy(
                    y_hbm.at[src_v.at[j0]], rows[0], gsem[0])
                for k in range(GRP):
                    b = k % 2
                    if k + 1 < GRP:
                        nb = 1 - b
                        descs[nb] = pltpu.async_copy(
                            y_hbm.at[src_v.at[j0 + k + 1]], rows[nb],
                            gsem[nb])
                    descs[b].wait()
                    pltpu.sync_copy(rows[b], agg_sh.at[dst_v.at[j0 + k]],
                                    add=True)
                return carry

            lax.fori_loop(0, NCHH // GRP, body, 0)
        plsc.subcore_barrier()
        pltpu.sync_copy(agg_sh.at[pl.ds(s * RPT, RPT)],
                        out_hbm.at[c, pl.ds(s * RPT, RPT)])

    return sc_deg, sc_msg


_R = 2000
_GRID = N_NODES // _R


def _tc_in(x, W0, b0r):
    def body(x_ref, w_ref, b_ref, o_ref):
        h = jnp.dot(x_ref[...], w_ref[...], preferred_element_type=jnp.float32)
        o_ref[...] = jnp.maximum(h + b_ref[...], 0.0)

    return pl.pallas_call(
        body,
        grid=(_GRID,),
        in_specs=[
            pl.BlockSpec((_R, D), lambda i: (i, 0)),
            pl.BlockSpec((D, D), lambda i: (0, 0)),
            pl.BlockSpec((1, D), lambda i: (0, 0)),
        ],
        out_specs=pl.BlockSpec((_R, D), lambda i: (i, 0)),
        out_shape=jax.ShapeDtypeStruct((N_NODES, D), jnp.float32),
    )(x, W0, b0r)


def _tc_dinv(degp):
    def body(p_ref, o_ref):
        deg = p_ref[0] + p_ref[1] + 1.0
        o_ref[...] = lax.rsqrt(deg)

    return pl.pallas_call(
        body,
        in_specs=[pl.BlockSpec((NC, N_PAD // 128, 128), lambda: (0, 0, 0))],
        out_specs=pl.BlockSpec((N_PAD // 128, 128), lambda: (0, 0)),
        out_shape=jax.ShapeDtypeStruct((N_PAD // 128, 128), jnp.float32),
    )(degp)


def _tc_scale_mm(h, W, dinv_col):
    def body(h_ref, w_ref, d_ref, o_ref):
        y = jnp.dot(h_ref[...], w_ref[...], preferred_element_type=jnp.float32)
        o_ref[...] = y * d_ref[...]

    return pl.pallas_call(
        body,
        grid=(_GRID,),
        in_specs=[
            pl.BlockSpec((_R, D), lambda i: (i, 0)),
            pl.BlockSpec((D, D), lambda i: (0, 0)),
            pl.BlockSpec((_R, 1), lambda i: (i, 0)),
        ],
        out_specs=pl.BlockSpec((_R, D), lambda i: (i, 0)),
        out_shape=jax.ShapeDtypeStruct((N_NODES, D), jnp.float32),
    )(h, W, dinv_col)


def _tc_mid(p, y, dinv_col, br, W):
    def body(p_ref, y_ref, d_ref, b_ref, w_ref, o_ref):
        agg = p_ref[0] + p_ref[1] + y_ref[...]
        h = jnp.maximum(agg * d_ref[...] + b_ref[...], 0.0)
        o_ref[...] = jnp.dot(h, w_ref[...],
                             preferred_element_type=jnp.float32) * d_ref[...]

    return pl.pallas_call(
        body,
        grid=(_GRID,),
        in_specs=[
            pl.BlockSpec((NC, _R, D), lambda i: (0, i, 0)),
            pl.BlockSpec((_R, D), lambda i: (i, 0)),
            pl.BlockSpec((_R, 1), lambda i: (i, 0)),
            pl.BlockSpec((1, D), lambda i: (0, 0)),
            pl.BlockSpec((D, D), lambda i: (0, 0)),
        ],
        out_specs=pl.BlockSpec((_R, D), lambda i: (i, 0)),
        out_shape=jax.ShapeDtypeStruct((N_NODES, D), jnp.float32),
    )(p, y, dinv_col, br, W)


def _tc_out(p, y, dinv_col, br, W, b3r):
    def body(p_ref, y_ref, d_ref, b_ref, w_ref, b3_ref, o_ref):
        agg = p_ref[0] + p_ref[1] + y_ref[...]
        h = jnp.maximum(agg * d_ref[...] + b_ref[...], 0.0)
        o_ref[...] = jnp.dot(h, w_ref[...],
                             preferred_element_type=jnp.float32) + b3_ref[...]

    return pl.pallas_call(
        body,
        grid=(_GRID,),
        in_specs=[
            pl.BlockSpec((NC, _R, D), lambda i: (0, i, 0)),
            pl.BlockSpec((_R, D), lambda i: (i, 0)),
            pl.BlockSpec((_R, 1), lambda i: (i, 0)),
            pl.BlockSpec((1, D), lambda i: (0, 0)),
            pl.BlockSpec((D, D), lambda i: (0, 0)),
            pl.BlockSpec((1, D), lambda i: (0, 0)),
        ],
        out_specs=pl.BlockSpec((_R, D), lambda i: (i, 0)),
        out_shape=jax.ShapeDtypeStruct((N_NODES, D), jnp.float32),
    )(p, y, dinv_col, br, W, b3r)


def kernel(x, edge_index, W0, b0, W1, b1, W2, b2, W3, b3):
    ei = edge_index.astype(jnp.int32)
    pad = E_PAD - E
    pad_src = jnp.arange(pad, dtype=jnp.int32) % N_NODES
    src3 = jnp.concatenate(
        [ei[0], pad_src]).reshape(NCH, NW, CHUNK).transpose(1, 0, 2)
    pad_dst = N_NODES + jnp.arange(pad, dtype=jnp.int32) % (N_PAD - N_NODES)
    dst3 = jnp.concatenate(
        [ei[1], pad_dst]).reshape(NCH, NW, CHUNK).transpose(1, 0, 2)

    zeros1 = jnp.zeros((N_PAD,), jnp.float32)
    zeros2 = jnp.zeros((N_PAD, D), jnp.float32)
    b0r = b0.reshape(1, D)
    b1r = b1.reshape(1, D)
    b2r = b2.reshape(1, D)
    b3r = b3.reshape(1, D)

    _sc_deg, _sc_msg = _sc_kernels()
    degp = _sc_deg(dst3, zeros1)
    h0 = _tc_in(x, W0, b0r)
    dinv2d = _tc_dinv(degp.reshape(NC, N_PAD // 128, 128))
    dinv_col = dinv2d.reshape(N_PAD, 1)[:N_NODES]

    y1 = _tc_scale_mm(h0, W1, dinv_col)
    p1 = _sc_msg(src3, dst3, y1, zeros2)
    y2 = _tc_mid(p1, y1, dinv_col, b1r, W2)
    p2 = _sc_msg(src3, dst3, y2, zeros2)
    logits = _tc_out(p2, y2, dinv_col, b2r, W3, b3r)
    return logits


# speedup vs baseline: 3.7253x; 1.0504x over previous
import functools

import jax
import jax.numpy as jnp
from jax import lax
from jax.experimental import pallas as pl
from jax.experimental.pallas import tpu as pltpu
from jax.experimental.pallas import tpu_sc as plsc

N_NODES = 10000
D = 128
E = 320000

NC = 2
NS = 16
NW = NC * NS
CHUNK = 128
EPW = E // NW
NCH = 80
EPW_PAD = NCH * CHUNK
E_PAD = EPW_PAD * NW
NCHH = NCH // 2
GRP = 20
N_PAD = 10240
DST_PAD = 10008
RPT = N_PAD // NS

@functools.cache
def _sc_kernels():
    mesh = plsc.VectorSubcoreMesh(core_axis_name="c", subcore_axis_name="s",
                                  num_cores=NC, num_subcores=NS)

    @functools.partial(
        pl.kernel,
        out_type=jax.ShapeDtypeStruct((NC, N_PAD), jnp.float32),
        mesh=mesh,
        scratch_types=[
            pltpu.VMEM((NCH, CHUNK), jnp.int32),
            pltpu.VMEM((CHUNK,), jnp.float32),
            pltpu.VMEM_SHARED((N_PAD,), jnp.float32),
        ],
    )
    def sc_deg(dst_hbm, zeros1_hbm, out_hbm, dst_v, ones_v, deg_sh):
        c = lax.axis_index("c")
        s = lax.axis_index("s")
        wid = s * NC + c
        for i in range(CHUNK // 16):
            ones_v[pl.ds(i * 16, 16)] = jnp.ones((16,), jnp.float32)
        pltpu.sync_copy(zeros1_hbm.at[pl.ds(s * RPT, RPT)],
                        deg_sh.at[pl.ds(s * RPT, RPT)])
        plsc.subcore_barrier()
        pltpu.sync_copy(dst_hbm.at[wid], dst_v)

        def body(j, carry):
            pltpu.sync_copy(ones_v, deg_sh.at[dst_v.at[j]], add=True)
            return carry

        lax.fori_loop(0, NCH, body, 0)
        plsc.subcore_barrier()
        pltpu.sync_copy(deg_sh.at[pl.ds(s * RPT, RPT)],
                        out_hbm.at[c, pl.ds(s * RPT, RPT)])

    @functools.partial(
        pl.kernel,
        out_type=jax.ShapeDtypeStruct((NC, N_PAD, D), jnp.float32),
        mesh=mesh,
        scratch_types=[
            pltpu.VMEM((NCHH, CHUNK), jnp.int32),
            pltpu.VMEM((NCHH, CHUNK), jnp.int32),
            [pltpu.VMEM((CHUNK, D), jnp.float32) for _ in range(2)],
            pltpu.VMEM_SHARED((N_PAD, D), jnp.float32),
            [pltpu.SemaphoreType.DMA for _ in range(2)],
        ],
    )
    def sc_msg(src_hbm, dst_hbm, y_hbm, zeros2_hbm, out_hbm,
               src_v, dst_v, rows, agg_sh, gsem):
        c = lax.axis_index("c")
        s = lax.axis_index("s")
        wid = s * NC + c
        pltpu.sync_copy(zeros2_hbm.at[pl.ds(s * RPT, RPT)],
                        agg_sh.at[pl.ds(s * RPT, RPT)])
        plsc.subcore_barrier()
        for half in range(2):
            pltpu.sync_copy(src_hbm.at[wid, pl.ds(half * NCHH, NCHH)],
                            src_v)
            pltpu.sync_copy(dst_hbm.at[wid, pl.ds(half * NCHH, NCHH)],
                            dst_v)

            def body(grp, carry):
                j0 = grp * GRP
                descs = [None, None]
                descs[0] = pltpu.async_copy(
                    y_hbm.at[src_v.at[j0]], rows[0], gsem[0])
                for k in range(GRP):
                    b = k % 2
                    if k + 1 < GRP:
                        nb = 1 - b
                        descs[nb] = pltpu.async_copy(
                            y_hbm.at[src_v.at[j0 + k + 1]], rows[nb],
                            gsem[nb])
                    descs[b].wait()
                    pltpu.sync_copy(rows[b], agg_sh.at[dst_v.at[j0 + k]],
                                    add=True)
                return carry

            lax.fori_loop(0, NCHH // GRP, body, 0)
        plsc.subcore_barrier()
        pltpu.sync_copy(agg_sh.at[pl.ds(s * RPT, RPT)],
                        out_hbm.at[c, pl.ds(s * RPT, RPT)])

    return sc_deg, sc_msg


_R = 2000
_GRID = N_NODES // _R


def _tc_in(x, W0, b0r):
    def body(x_ref, w_ref, b_ref, o_ref):
        h = jnp.dot(x_ref[...], w_ref[...], preferred_element_type=jnp.float32)
        o_ref[...] = jnp.maximum(h + b_ref[...], 0.0)

    return pl.pallas_call(
        body,
        grid=(_GRID,),
        in_specs=[
            pl.BlockSpec((_R, D), lambda i: (i, 0)),
            pl.BlockSpec((D, D), lambda i: (0, 0)),
            pl.BlockSpec((1, D), lambda i: (0, 0)),
        ],
        out_specs=pl.BlockSpec((_R, D), lambda i: (i, 0)),
        out_shape=jax.ShapeDtypeStruct((N_NODES, D), jnp.float32),
    )(x, W0, b0r)


def _tc_dinv(degp):
    def body(p_ref, o_ref):
        deg = p_ref[0] + p_ref[1] + 1.0
        o_ref[...] = lax.rsqrt(deg)

    return pl.pallas_call(
        body,
        in_specs=[pl.BlockSpec((NC, N_PAD // 128, 128), lambda: (0, 0, 0))],
        out_specs=pl.BlockSpec((N_PAD // 128, 128), lambda: (0, 0)),
        out_shape=jax.ShapeDtypeStruct((N_PAD // 128, 128), jnp.float32),
    )(degp)


def _tc_scale_mm(h, W, dinv_col):
    def body(h_ref, w_ref, d_ref, o_ref):
        y = jnp.dot(h_ref[...], w_ref[...], preferred_element_type=jnp.float32)
        o_ref[...] = y * d_ref[...]

    return pl.pallas_call(
        body,
        grid=(_GRID,),
        in_specs=[
            pl.BlockSpec((_R, D), lambda i: (i, 0)),
            pl.BlockSpec((D, D), lambda i: (0, 0)),
            pl.BlockSpec((_R, 1), lambda i: (i, 0)),
        ],
        out_specs=pl.BlockSpec((_R, D), lambda i: (i, 0)),
        out_shape=jax.ShapeDtypeStruct((N_NODES, D), jnp.float32),
    )(h, W, dinv_col)


def _tc_mid(p, y, dinv_col, br, W):
    def body(p_ref, y_ref, d_ref, b_ref, w_ref, o_ref):
        agg = p_ref[0] + p_ref[1] + y_ref[...]
        h = jnp.maximum(agg * d_ref[...] + b_ref[...], 0.0)
        o_ref[...] = jnp.dot(h, w_ref[...],
                             preferred_element_type=jnp.float32) * d_ref[...]

    return pl.pallas_call(
        body,
        grid=(_GRID,),
        in_specs=[
            pl.BlockSpec((NC, _R, D), lambda i: (0, i, 0)),
            pl.BlockSpec((_R, D), lambda i: (i, 0)),
            pl.BlockSpec((_R, 1), lambda i: (i, 0)),
            pl.BlockSpec((1, D), lambda i: (0, 0)),
            pl.BlockSpec((D, D), lambda i: (0, 0)),
        ],
        out_specs=pl.BlockSpec((_R, D), lambda i: (i, 0)),
        out_shape=jax.ShapeDtypeStruct((N_NODES, D), jnp.float32),
    )(p, y, dinv_col, br, W)


def _tc_out(p, y, dinv_col, br, W, b3r):
    def body(p_ref, y_ref, d_ref, b_ref, w_ref, b3_ref, o_ref):
        agg = p_ref[0] + p_ref[1] + y_ref[...]
        h = jnp.maximum(agg * d_ref[...] + b_ref[...], 0.0)
        o_ref[...] = jnp.dot(h, w_ref[...],
                             preferred_element_type=jnp.float32) + b3_ref[...]

    return pl.pallas_call(
        body,
        grid=(_GRID,),
        in_specs=[
            pl.BlockSpec((NC, _R, D), lambda i: (0, i, 0)),
            pl.BlockSpec((_R, D), lambda i: (i, 0)),
            pl.BlockSpec((_R, 1), lambda i: (i, 0)),
            pl.BlockSpec((1, D), lambda i: (0, 0)),
            pl.BlockSpec((D, D), lambda i: (0, 0)),
            pl.BlockSpec((1, D), lambda i: (0, 0)),
        ],
        out_specs=pl.BlockSpec((_R, D), lambda i: (i, 0)),
        out_shape=jax.ShapeDtypeStruct((N_NODES, D), jnp.float32),
    )(p, y, dinv_col, br, W, b3r)


def kernel(x, edge_index, W0, b0, W1, b1, W2, b2, W3, b3):
    ei = edge_index.astype(jnp.int32)
    pad = E_PAD - E
    pad_src = jnp.arange(pad, dtype=jnp.int32) % N_NODES
    src3 = jnp.concatenate(
        [ei[0], pad_src]).reshape(NCH, NW, CHUNK).transpose(1, 0, 2)
    pad_dst = N_NODES + jnp.arange(pad, dtype=jnp.int32) % (N_PAD - N_NODES)
    dst3 = jnp.concatenate(
        [ei[1], pad_dst]).reshape(NCH, NW, CHUNK).transpose(1, 0, 2)

    zeros1 = jnp.zeros((N_PAD,), jnp.float32)
    zeros2 = jnp.zeros((N_PAD, D), jnp.float32)
    b0r = b0.reshape(1, D)
    b1r = b1.reshape(1, D)
    b2r = b2.reshape(1, D)
    b3r = b3.reshape(1, D)

    _sc_deg, _sc_msg = _sc_kernels()
    degp = _sc_deg(dst3, zeros1)
    h0 = _tc_in(x, W0, b0r)
    dinv2d = _tc_dinv(degp.reshape(NC, N_PAD // 128, 128))
    dinv_col = dinv2d.reshape(N_PAD, 1)[:N_NODES]

    y1 = _tc_scale_mm(h0, W1, dinv_col)
    p1 = _sc_msg(src3, dst3, y1, zeros2)
    y2 = _tc_mid(p1, y1, dinv_col, b1r, W2)
    p2 = _sc_msg(src3, dst3, y2, zeros2)
    logits = _tc_out(p2, y2, dinv_col, b2r, W3, b3r)
    return logits


# speedup vs baseline: 3.7735x; 1.0129x over previous
import functools

import jax
import jax.numpy as jnp
from jax import lax
from jax.experimental import pallas as pl
from jax.experimental.pallas import tpu as pltpu
from jax.experimental.pallas import tpu_sc as plsc

N_NODES = 10000
D = 128
E = 320000

NC = 2
NS = 16
NW = NC * NS
CHUNK = 128
EPW = E // NW
NCH = 80
EPW_PAD = NCH * CHUNK
E_PAD = EPW_PAD * NW
NCHH = NCH // 2
GRP = 40
N_PAD = 10240
DST_PAD = 10008
RPT = N_PAD // NS

@functools.cache
def _sc_kernels():
    mesh = plsc.VectorSubcoreMesh(core_axis_name="c", subcore_axis_name="s",
                                  num_cores=NC, num_subcores=NS)

    @functools.partial(
        pl.kernel,
        out_type=jax.ShapeDtypeStruct((NC, N_PAD), jnp.float32),
        mesh=mesh,
        scratch_types=[
            pltpu.VMEM((NCH, CHUNK), jnp.int32),
            pltpu.VMEM((CHUNK,), jnp.float32),
            pltpu.VMEM_SHARED((N_PAD,), jnp.float32),
        ],
    )
    def sc_deg(dst_hbm, zeros1_hbm, out_hbm, dst_v, ones_v, deg_sh):
        c = lax.axis_index("c")
        s = lax.axis_index("s")
        wid = s * NC + c
        for i in range(CHUNK // 16):
            ones_v[pl.ds(i * 16, 16)] = jnp.ones((16,), jnp.float32)
        pltpu.sync_copy(zeros1_hbm.at[pl.ds(s * RPT, RPT)],
                        deg_sh.at[pl.ds(s * RPT, RPT)])
        plsc.subcore_barrier()
        pltpu.sync_copy(dst_hbm.at[wid], dst_v)

        def body(j, carry):
            pltpu.sync_copy(ones_v, deg_sh.at[dst_v.at[j]], add=True)
            return carry

        lax.fori_loop(0, NCH, body, 0)
        plsc.subcore_barrier()
        pltpu.sync_copy(deg_sh.at[pl.ds(s * RPT, RPT)],
                        out_hbm.at[c, pl.ds(s * RPT, RPT)])

    @functools.partial(
        pl.kernel,
        out_type=jax.ShapeDtypeStruct((NC, N_PAD, D), jnp.float32),
        mesh=mesh,
        scratch_types=[
            pltpu.VMEM((NCHH, CHUNK), jnp.int32),
            pltpu.VMEM((NCHH, CHUNK), jnp.int32),
            [pltpu.VMEM((CHUNK, D), jnp.float32) for _ in range(2)],
            pltpu.VMEM_SHARED((N_PAD, D), jnp.float32),
            [pltpu.SemaphoreType.DMA for _ in range(2)],
        ],
    )
    def sc_msg(src_hbm, dst_hbm, y_hbm, zeros2_hbm, out_hbm,
               src_v, dst_v, rows, agg_sh, gsem):
        c = lax.axis_index("c")
        s = lax.axis_index("s")
        wid = s * NC + c
        pltpu.sync_copy(zeros2_hbm.at[pl.ds(s * RPT, RPT)],
                        agg_sh.at[pl.ds(s * RPT, RPT)])
        plsc.subcore_barrier()
        for half in range(2):
            pltpu.sync_copy(src_hbm.at[wid, pl.ds(half * NCHH, NCHH)],
                            src_v)
            pltpu.sync_copy(dst_hbm.at[wid, pl.ds(half * NCHH, NCHH)],
                            dst_v)

            def body(grp, carry):
                j0 = grp * GRP
                descs = [None, None]
                descs[0] = pltpu.async_copy(
                    y_hbm.at[src_v.at[j0]], rows[0], gsem[0])
                for k in range(GRP):
                    b = k % 2
                    if k + 1 < GRP:
                        nb = 1 - b
                        descs[nb] = pltpu.async_copy(
                            y_hbm.at[src_v.at[j0 + k + 1]], rows[nb],
                            gsem[nb])
                    descs[b].wait()
                    pltpu.sync_copy(rows[b], agg_sh.at[dst_v.at[j0 + k]],
                                    add=True)
                return carry

            lax.fori_loop(0, NCHH // GRP, body, 0)
        plsc.subcore_barrier()
        pltpu.sync_copy(agg_sh.at[pl.ds(s * RPT, RPT)],
                        out_hbm.at[c, pl.ds(s * RPT, RPT)])

    return sc_deg, sc_msg


_R = 2000
_GRID = N_NODES // _R


def _tc_in(x, W0, b0r):
    def body(x_ref, w_ref, b_ref, o_ref):
        h = jnp.dot(x_ref[...], w_ref[...], preferred_element_type=jnp.float32)
        o_ref[...] = jnp.maximum(h + b_ref[...], 0.0)

    return pl.pallas_call(
        body,
        grid=(_GRID,),
        in_specs=[
            pl.BlockSpec((_R, D), lambda i: (i, 0)),
            pl.BlockSpec((D, D), lambda i: (0, 0)),
            pl.BlockSpec((1, D), lambda i: (0, 0)),
        ],
        out_specs=pl.BlockSpec((_R, D), lambda i: (i, 0)),
        out_shape=jax.ShapeDtypeStruct((N_NODES, D), jnp.float32),
    )(x, W0, b0r)


def _tc_dinv(degp):
    def body(p_ref, o_ref):
        deg = p_ref[0] + p_ref[1] + 1.0
        o_ref[...] = lax.rsqrt(deg)

    return pl.pallas_call(
        body,
        in_specs=[pl.BlockSpec((NC, N_PAD // 128, 128), lambda: (0, 0, 0))],
        out_specs=pl.BlockSpec((N_PAD // 128, 128), lambda: (0, 0)),
        out_shape=jax.ShapeDtypeStruct((N_PAD // 128, 128), jnp.float32),
    )(degp)


def _tc_scale_mm(h, W, dinv_col):
    def body(h_ref, w_ref, d_ref, o_ref):
        y = jnp.dot(h_ref[...], w_ref[...], preferred_element_type=jnp.float32)
        o_ref[...] = y * d_ref[...]

    return pl.pallas_call(
        body,
        grid=(_GRID,),
        in_specs=[
            pl.BlockSpec((_R, D), lambda i: (i, 0)),
            pl.BlockSpec((D, D), lambda i: (0, 0)),
            pl.BlockSpec((_R, 1), lambda i: (i, 0)),
        ],
        out_specs=pl.BlockSpec((_R, D), lambda i: (i, 0)),
        out_shape=jax.ShapeDtypeStruct((N_NODES, D), jnp.float32),
    )(h, W, dinv_col)


def _tc_mid(p, y, dinv_col, br, W):
    def body(p_ref, y_ref, d_ref, b_ref, w_ref, o_ref):
        agg = p_ref[0] + p_ref[1] + y_ref[...]
        h = jnp.maximum(agg * d_ref[...] + b_ref[...], 0.0)
        o_ref[...] = jnp.dot(h, w_ref[...],
                             preferred_element_type=jnp.float32) * d_ref[...]

    return pl.pallas_call(
        body,
        grid=(_GRID,),
        in_specs=[
            pl.BlockSpec((NC, _R, D), lambda i: (0, i, 0)),
            pl.BlockSpec((_R, D), lambda i: (i, 0)),
            pl.BlockSpec((_R, 1), lambda i: (i, 0)),
            pl.BlockSpec((1, D), lambda i: (0, 0)),
            pl.BlockSpec((D, D), lambda i: (0, 0)),
        ],
        out_specs=pl.BlockSpec((_R, D), lambda i: (i, 0)),
        out_shape=jax.ShapeDtypeStruct((N_NODES, D), jnp.float32),
    )(p, y, dinv_col, br, W)


def _tc_out(p, y, dinv_col, br, W, b3r):
    def body(p_ref, y_ref, d_ref, b_ref, w_ref, b3_ref, o_ref):
        agg = p_ref[0] + p_ref[1] + y_ref[...]
        h = jnp.maximum(agg * d_ref[...] + b_ref[...], 0.0)
        o_ref[...] = jnp.dot(h, w_ref[...],
                             preferred_element_type=jnp.float32) + b3_ref[...]

    return pl.pallas_call(
        body,
        grid=(_GRID,),
        in_specs=[
            pl.BlockSpec((NC, _R, D), lambda i: (0, i, 0)),
            pl.BlockSpec((_R, D), lambda i: (i, 0)),
            pl.BlockSpec((_R, 1), lambda i: (i, 0)),
            pl.BlockSpec((1, D), lambda i: (0, 0)),
            pl.BlockSpec((D, D), lambda i: (0, 0)),
            pl.BlockSpec((1, D), lambda i: (0, 0)),
        ],
        out_specs=pl.BlockSpec((_R, D), lambda i: (i, 0)),
        out_shape=jax.ShapeDtypeStruct((N_NODES, D), jnp.float32),
    )(p, y, dinv_col, br, W, b3r)


def kernel(x, edge_index, W0, b0, W1, b1, W2, b2, W3, b3):
    ei = edge_index.astype(jnp.int32)
    pad = E_PAD - E
    pad_src = jnp.arange(pad, dtype=jnp.int32) % N_NODES
    src3 = jnp.concatenate(
        [ei[0], pad_src]).reshape(NCH, NW, CHUNK).transpose(1, 0, 2)
    pad_dst = N_NODES + jnp.arange(pad, dtype=jnp.int32) % (N_PAD - N_NODES)
    dst3 = jnp.concatenate(
        [ei[1], pad_dst]).reshape(NCH, NW, CHUNK).transpose(1, 0, 2)

    zeros1 = jnp.zeros((N_PAD,), jnp.float32)
    zeros2 = jnp.zeros((N_PAD, D), jnp.float32)
    b0r = b0.reshape(1, D)
    b1r = b1.reshape(1, D)
    b2r = b2.reshape(1, D)
    b3r = b3.reshape(1, D)

    _sc_deg, _sc_msg = _sc_kernels()
    degp = _sc_deg(dst3, zeros1)
    h0 = _tc_in(x, W0, b0r)
    dinv2d = _tc_dinv(degp.reshape(NC, N_PAD // 128, 128))
    dinv_col = dinv2d.reshape(N_PAD, 1)[:N_NODES]

    y1 = _tc_scale_mm(h0, W1, dinv_col)
    p1 = _sc_msg(src3, dst3, y1, zeros2)
    y2 = _tc_mid(p1, y1, dinv_col, b1r, W2)
    p2 = _sc_msg(src3, dst3, y2, zeros2)
    logits = _tc_out(p2, y2, dinv_col, b2r, W3, b3r)
    return logits
